# Initial kernel scaffold; baseline (speedup 1.0000x reference)
import functools

import jax
import jax.numpy as jnp
from jax import lax
from jax.experimental import pallas as pl
from jax.experimental.pallas import tpu as pltpu
from jax.experimental.pallas import tpu_sc as plsc

N = 10000
E = 320000
D = 128
ET = 16
NPAD = 10240
NC, NS = 2, 16
NW = NC * NS
EPW = E // NW
CH = 80
NCH = EPW // CH
RPW = NPAD // NS
_F32 = jnp.float32


def _mesh():
    return plsc.VectorSubcoreMesh(core_axis_name="c", subcore_axis_name="s")


def _deg_sc(dst_i):
    @functools.partial(
        pl.kernel,
        out_type=jax.ShapeDtypeStruct((NC, NPAD), _F32),
        mesh=_mesh(),
        scratch_types=[
            pltpu.VMEM((CH,), jnp.int32),
            pltpu.VMEM((CH,), _F32),
            pltpu.VMEM((RPW,), _F32),
            pltpu.VMEM_SHARED((NPAD,), _F32),
        ],
    )
    def k(dst_hbm, out_hbm, idx_v, ones_v, zrow_v, acc_sh):
        c = lax.axis_index("c")
        s = lax.axis_index("s")
        wid = s * NC + c
        zero16 = jnp.zeros((16,), _F32)
        one16 = jnp.ones((16,), _F32)
        for j in range(RPW // 16):
            zrow_v[pl.ds(j * 16, 16)] = zero16
        for j in range(CH // 16):
            ones_v[pl.ds(j * 16, 16)] = one16
        pltpu.sync_copy(zrow_v, acc_sh.at[pl.ds(s * RPW, RPW)])
        plsc.subcore_barrier()
        base = wid * EPW

        @pl.loop(0, NCH)
        def _(i):
            pltpu.sync_copy(dst_hbm.at[pl.ds(base + i * CH, CH)], idx_v)
            pltpu.sync_copy(ones_v, acc_sh.at[idx_v], add=True)

        plsc.subcore_barrier()
        pltpu.sync_copy(acc_sh.at[pl.ds(s * RPW, RPW)],
                        out_hbm.at[c, pl.ds(s * RPW, RPW)])

    return k(dst_i)


def _segsum_sc(table, src_i, dst_i):
    @functools.partial(
        pl.kernel,
        out_type=jax.ShapeDtypeStruct((NC, NPAD, D), _F32),
        mesh=_mesh(),
        scratch_types=[
            pltpu.VMEM((CH,), jnp.int32),
            pltpu.VMEM((CH,), jnp.int32),
            pltpu.VMEM((CH, D), _F32),
            pltpu.VMEM_SHARED((NPAD, D), _F32),
        ],
    )
    def k(table_hbm, src_hbm, dst_hbm, out_hbm, sidx_v, didx_v, rows_v, acc_sh):
        c = lax.axis_index("c")
        s = lax.axis_index("s")
        wid = s * NC + c
        zero16 = jnp.zeros((16,), _F32)

        @pl.loop(0, CH)
        def _(j):
            for t in range(D // 16):
                rows_v[j, pl.ds(t * 16, 16)] = zero16

        for t in range(RPW // CH):
            pltpu.sync_copy(rows_v, acc_sh.at[pl.ds(s * RPW + t * CH, CH)])
        plsc.subcore_barrier()
        base = wid * EPW

        @pl.loop(0, NCH)
        def _(i):
            pltpu.sync_copy(src_hbm.at[pl.ds(base + i * CH, CH)], sidx_v)
            pltpu.sync_copy(dst_hbm.at[pl.ds(base + i * CH, CH)], didx_v)
            pltpu.sync_copy(table_hbm.at[sidx_v], rows_v)
            pltpu.sync_copy(rows_v, acc_sh.at[didx_v], add=True)

        plsc.subcore_barrier()
        pltpu.sync_copy(acc_sh.at[pl.ds(s * RPW, RPW)],
                        out_hbm.at[c, pl.ds(s * RPW, RPW)])

    return k(table, src_i, dst_i)


def _edge_mix_sc(p1, p2, src_i, dst_i):
    @functools.partial(
        pl.kernel,
        out_type=jax.ShapeDtypeStruct((E, ET), _F32),
        mesh=_mesh(),
        scratch_types=[
            pltpu.VMEM((CH,), jnp.int32),
            pltpu.VMEM((CH,), jnp.int32),
            pltpu.VMEM((CH, ET), _F32),
            pltpu.VMEM((CH, ET), _F32),
        ],
    )
    def k(p1_hbm, p2_hbm, src_hbm, dst_hbm, out_hbm, sidx_v, didx_v, a_v, b_v):
        c = lax.axis_index("c")
        s = lax.axis_index("s")
        wid = s * NC + c
        base = wid * EPW

        @pl.loop(0, NCH)
        def _(i):
            pltpu.sync_copy(src_hbm.at[pl.ds(base + i * CH, CH)], sidx_v)
            pltpu.sync_copy(dst_hbm.at[pl.ds(base + i * CH, CH)], didx_v)
            pltpu.sync_copy(p1_hbm.at[sidx_v], a_v)
            pltpu.sync_copy(p2_hbm.at[didx_v], b_v)

            @pl.loop(0, CH)
            def _(j):
                a_v[j] = a_v[j] + b_v[j]

            pltpu.sync_copy(a_v, out_hbm.at[pl.ds(base + i * CH, CH)])

    return k(p1, p2, src_i, dst_i)


_BM = 1024


def _dot(a, b):
    return lax.dot_general(a, b, (((1,), (0,)), ((), ())),
                           precision=lax.Precision.HIGHEST,
                           preferred_element_type=_F32)


def _mm_tc(x, w):
    k_dim = w.shape[1]

    def body(x_ref, w_ref, o_ref):
        o_ref[...] = _dot(x_ref[...], w_ref[...])

    return pl.pallas_call(
        body,
        grid=(NPAD // _BM,),
        in_specs=[pl.BlockSpec((_BM, D), lambda i: (i, 0)),
                  pl.BlockSpec((D, k_dim), lambda i: (0, 0))],
        out_specs=pl.BlockSpec((_BM, k_dim), lambda i: (i, 0)),
        out_shape=jax.ShapeDtypeStruct((NPAD, k_dim), _F32),
    )(x, w)


def _rscale_tc(d0, d1, s1):
    def body(d0_ref, d1_ref, s_ref, r_ref, o_ref):
        deg = jnp.maximum(d0_ref[...] + d1_ref[...], 1.0)
        r = lax.rsqrt(deg)
        r_ref[...] = r
        o_ref[...] = s_ref[...] * r

    return pl.pallas_call(
        body,
        grid=(NPAD // _BM,),
        in_specs=[pl.BlockSpec((_BM, 1), lambda i: (i, 0)),
                  pl.BlockSpec((_BM, 1), lambda i: (i, 0)),
                  pl.BlockSpec((_BM, D), lambda i: (i, 0))],
        out_specs=[pl.BlockSpec((_BM, 1), lambda i: (i, 0)),
                   pl.BlockSpec((_BM, D), lambda i: (i, 0))],
        out_shape=[jax.ShapeDtypeStruct((NPAD, 1), _F32),
                   jax.ShapeDtypeStruct((NPAD, D), _F32)],
    )(d0, d1, s1)


def _layer_mid_tc(q0, q1, r, b, w):
    def body(q0_ref, q1_ref, r_ref, b_ref, w_ref, o_ref):
        h = jnp.maximum((q0_ref[...] + q1_ref[...]) * r_ref[...] + b_ref[...],
                        0.0)
        o_ref[...] = _dot(h, w_ref[...]) * r_ref[...]

    return pl.pallas_call(
        body,
        grid=(NPAD // _BM,),
        in_specs=[pl.BlockSpec((_BM, D), lambda i: (i, 0)),
                  pl.BlockSpec((_BM, D), lambda i: (i, 0)),
                  pl.BlockSpec((_BM, 1), lambda i: (i, 0)),
                  pl.BlockSpec((1, D), lambda i: (0, 0)),
                  pl.BlockSpec((D, D), lambda i: (0, 0))],
        out_specs=pl.BlockSpec((_BM, D), lambda i: (i, 0)),
        out_shape=jax.ShapeDtypeStruct((NPAD, D), _F32),
    )(q0, q1, r, b, w)


def _layer_out_tc(q0, q1, r, b, wa, wb, bfc):
    def body(q0_ref, q1_ref, r_ref, b_ref, wa_ref, wb_ref, bfc_ref,
             p1_ref, p2_ref):
        h = jnp.maximum((q0_ref[...] + q1_ref[...]) * r_ref[...] + b_ref[...],
                        0.0)
        p1_ref[...] = _dot(h, wa_ref[...]) + bfc_ref[...]
        p2_ref[...] = _dot(h, wb_ref[...])

    return pl.pallas_call(
        body,
        grid=(NPAD // _BM,),
        in_specs=[pl.BlockSpec((_BM, D), lambda i: (i, 0)),
                  pl.BlockSpec((_BM, D), lambda i: (i, 0)),
                  pl.BlockSpec((_BM, 1), lambda i: (i, 0)),
                  pl.BlockSpec((1, D), lambda i: (0, 0)),
                  pl.BlockSpec((D, ET), lambda i: (0, 0)),
                  pl.BlockSpec((D, ET), lambda i: (0, 0)),
                  pl.BlockSpec((1, ET), lambda i: (0, 0))],
        out_specs=[pl.BlockSpec((_BM, ET), lambda i: (i, 0)),
                   pl.BlockSpec((_BM, ET), lambda i: (i, 0))],
        out_shape=[jax.ShapeDtypeStruct((NPAD, ET), _F32),
                   jax.ShapeDtypeStruct((NPAD, ET), _F32)],
    )(q0, q1, r, b, wa, wb, bfc)


def kernel(x, edges, W1, b1, W2, b2, Wfc, bfc):
    src = edges[0]
    dst = edges[1]
    xp = jnp.zeros((NPAD, D), _F32).at[:N].set(x)

    degp = _deg_sc(dst)
    s1 = _mm_tc(xp, W1)

    d0 = degp[0].reshape(NPAD, 1)
    d1 = degp[1].reshape(NPAD, 1)
    r, s1s = _rscale_tc(d0, d1, s1)

    qp = _segsum_sc(s1s, src, dst)
    s2s = _layer_mid_tc(qp[0], qp[1], r, b1.reshape(1, D), W2)

    qp2 = _segsum_sc(s2s, src, dst)
    p1, p2 = _layer_out_tc(qp2[0], qp2[1], r, b2.reshape(1, D),
                           Wfc[:D], Wfc[D:], bfc.reshape(1, ET))

    return _edge_mix_sc(p1, p2, src, dst)


# speedup vs baseline: 7.5118x; 7.5118x over previous
import functools

import jax
import jax.numpy as jnp
from jax import lax
from jax.experimental import pallas as pl
from jax.experimental.pallas import tpu as pltpu
from jax.experimental.pallas import tpu_sc as plsc

N = 10000
E = 320000
D = 128
ET = 16
NPAD = 10240
NC, NS = 2, 16
NW = NC * NS
EPW = E // NW
CH = 80
NCH = EPW // CH
RPW = NPAD // NS
_F32 = jnp.float32


def _mesh():
    return plsc.VectorSubcoreMesh(core_axis_name="c", subcore_axis_name="s")


def _deg_sc(dst_i):
    @functools.partial(
        pl.kernel,
        out_type=jax.ShapeDtypeStruct((NC, NPAD), _F32),
        mesh=_mesh(),
        scratch_types=[
            pltpu.VMEM((CH,), jnp.int32),
            pltpu.VMEM((CH,), _F32),
            pltpu.VMEM((RPW,), _F32),
            pltpu.VMEM_SHARED((NPAD,), _F32),
        ],
    )
    def k(dst_hbm, out_hbm, idx_v, ones_v, zrow_v, acc_sh):
        c = lax.axis_index("c")
        s = lax.axis_index("s")
        wid = s * NC + c
        zero16 = jnp.zeros((16,), _F32)
        one16 = jnp.ones((16,), _F32)
        for j in range(RPW // 16):
            zrow_v[pl.ds(j * 16, 16)] = zero16
        for j in range(CH // 16):
            ones_v[pl.ds(j * 16, 16)] = one16
        pltpu.sync_copy(zrow_v, acc_sh.at[pl.ds(s * RPW, RPW)])
        plsc.subcore_barrier()
        base = wid * EPW

        @pl.loop(0, NCH)
        def _(i):
            pltpu.sync_copy(dst_hbm.at[pl.ds(base + i * CH, CH)], idx_v)
            pltpu.sync_copy(ones_v, acc_sh.at[idx_v], add=True)

        plsc.subcore_barrier()
        pltpu.sync_copy(acc_sh.at[pl.ds(s * RPW, RPW)],
                        out_hbm.at[c, pl.ds(s * RPW, RPW)])

    return k(dst_i)


def _segsum_sc(table, src_i, dst_i):
    @functools.partial(
        pl.kernel,
        out_type=jax.ShapeDtypeStruct((NC, NPAD, D), _F32),
        mesh=_mesh(),
        scratch_types=[
            pltpu.VMEM((CH,), jnp.int32),
            pltpu.VMEM((CH,), jnp.int32),
            pltpu.VMEM((CH, D), _F32),
            pltpu.VMEM_SHARED((NPAD, D), _F32),
        ],
    )
    def k(table_hbm, src_hbm, dst_hbm, out_hbm, sidx_v, didx_v, rows_v, acc_sh):
        c = lax.axis_index("c")
        s = lax.axis_index("s")
        wid = s * NC + c
        zero16 = jnp.zeros((16,), _F32)

        @pl.loop(0, CH)
        def _(j):
            for t in range(D // 16):
                rows_v[j, pl.ds(t * 16, 16)] = zero16

        for t in range(RPW // CH):
            pltpu.sync_copy(rows_v, acc_sh.at[pl.ds(s * RPW + t * CH, CH)])
        plsc.subcore_barrier()
        base = wid * EPW

        @pl.loop(0, NCH)
        def _(i):
            pltpu.sync_copy(src_hbm.at[pl.ds(base + i * CH, CH)], sidx_v)
            pltpu.sync_copy(dst_hbm.at[pl.ds(base + i * CH, CH)], didx_v)
            pltpu.sync_copy(table_hbm.at[sidx_v], rows_v)
            pltpu.sync_copy(rows_v, acc_sh.at[didx_v], add=True)

        plsc.subcore_barrier()
        pltpu.sync_copy(acc_sh.at[pl.ds(s * RPW, RPW)],
                        out_hbm.at[c, pl.ds(s * RPW, RPW)])

    return k(table, src_i, dst_i)


def _edge_mix_sc(p1, p2, src_i, dst_i):
    @functools.partial(
        pl.kernel,
        out_type=jax.ShapeDtypeStruct((E, ET), _F32),
        mesh=_mesh(),
        compiler_params=pltpu.CompilerParams(use_tc_tiling_on_sc=False),
        scratch_types=[
            pltpu.VMEM((CH,), jnp.int32),
            pltpu.VMEM((CH,), jnp.int32),
            pltpu.VMEM((CH, ET), _F32),
            pltpu.VMEM((CH, ET), _F32),
        ],
    )
    def k(p1_hbm, p2_hbm, src_hbm, dst_hbm, out_hbm, sidx_v, didx_v, a_v, b_v):
        c = lax.axis_index("c")
        s = lax.axis_index("s")
        wid = s * NC + c
        base = wid * EPW

        @pl.loop(0, NCH)
        def _(i):
            pltpu.sync_copy(src_hbm.at[pl.ds(base + i * CH, CH)], sidx_v)
            pltpu.sync_copy(dst_hbm.at[pl.ds(base + i * CH, CH)], didx_v)
            pltpu.sync_copy(p1_hbm.at[sidx_v], a_v)
            pltpu.sync_copy(p2_hbm.at[didx_v], b_v)

            @pl.loop(0, CH)
            def _(j):
                a_v[j] = a_v[j] + b_v[j]

            pltpu.sync_copy(a_v, out_hbm.at[pl.ds(base + i * CH, CH)])

    return k(p1, p2, src_i, dst_i)


_BM = 1024


def _dot(a, b):
    return lax.dot_general(a, b, (((1,), (0,)), ((), ())),
                           precision=lax.Precision.HIGHEST,
                           preferred_element_type=_F32)


def _mm_tc(x, w):
    k_dim = w.shape[1]

    def body(x_ref, w_ref, o_ref):
        o_ref[...] = _dot(x_ref[...], w_ref[...])

    return pl.pallas_call(
        body,
        grid=(NPAD // _BM,),
        in_specs=[pl.BlockSpec((_BM, D), lambda i: (i, 0)),
                  pl.BlockSpec((D, k_dim), lambda i: (0, 0))],
        out_specs=pl.BlockSpec((_BM, k_dim), lambda i: (i, 0)),
        out_shape=jax.ShapeDtypeStruct((NPAD, k_dim), _F32),
    )(x, w)


def _rscale_tc(d0, d1, s1):
    def body(d0_ref, d1_ref, s_ref, r_ref, o_ref):
        deg = jnp.maximum(d0_ref[...] + d1_ref[...], 1.0)
        r = lax.rsqrt(deg)
        r_ref[...] = r
        o_ref[...] = s_ref[...] * r

    return pl.pallas_call(
        body,
        grid=(NPAD // _BM,),
        in_specs=[pl.BlockSpec((_BM, 1), lambda i: (i, 0)),
                  pl.BlockSpec((_BM, 1), lambda i: (i, 0)),
                  pl.BlockSpec((_BM, D), lambda i: (i, 0))],
        out_specs=[pl.BlockSpec((_BM, 1), lambda i: (i, 0)),
                   pl.BlockSpec((_BM, D), lambda i: (i, 0))],
        out_shape=[jax.ShapeDtypeStruct((NPAD, 1), _F32),
                   jax.ShapeDtypeStruct((NPAD, D), _F32)],
    )(d0, d1, s1)


def _layer_mid_tc(q0, q1, r, b, w):
    def body(q0_ref, q1_ref, r_ref, b_ref, w_ref, o_ref):
        h = jnp.maximum((q0_ref[...] + q1_ref[...]) * r_ref[...] + b_ref[...],
                        0.0)
        o_ref[...] = _dot(h, w_ref[...]) * r_ref[...]

    return pl.pallas_call(
        body,
        grid=(NPAD // _BM,),
        in_specs=[pl.BlockSpec((_BM, D), lambda i: (i, 0)),
                  pl.BlockSpec((_BM, D), lambda i: (i, 0)),
                  pl.BlockSpec((_BM, 1), lambda i: (i, 0)),
                  pl.BlockSpec((1, D), lambda i: (0, 0)),
                  pl.BlockSpec((D, D), lambda i: (0, 0))],
        out_specs=pl.BlockSpec((_BM, D), lambda i: (i, 0)),
        out_shape=jax.ShapeDtypeStruct((NPAD, D), _F32),
    )(q0, q1, r, b, w)


def _layer_out_tc(q0, q1, r, b, wa, wb, bfc):
    def body(q0_ref, q1_ref, r_ref, b_ref, wa_ref, wb_ref, bfc_ref,
             p1_ref, p2_ref):
        h = jnp.maximum((q0_ref[...] + q1_ref[...]) * r_ref[...] + b_ref[...],
                        0.0)
        p1_ref[...] = _dot(h, wa_ref[...]) + bfc_ref[...]
        p2_ref[...] = _dot(h, wb_ref[...])

    return pl.pallas_call(
        body,
        grid=(NPAD // _BM,),
        in_specs=[pl.BlockSpec((_BM, D), lambda i: (i, 0)),
                  pl.BlockSpec((_BM, D), lambda i: (i, 0)),
                  pl.BlockSpec((_BM, 1), lambda i: (i, 0)),
                  pl.BlockSpec((1, D), lambda i: (0, 0)),
                  pl.BlockSpec((D, ET), lambda i: (0, 0)),
                  pl.BlockSpec((D, ET), lambda i: (0, 0)),
                  pl.BlockSpec((1, ET), lambda i: (0, 0))],
        out_specs=[pl.BlockSpec((_BM, ET), lambda i: (i, 0)),
                   pl.BlockSpec((_BM, ET), lambda i: (i, 0))],
        out_shape=[jax.ShapeDtypeStruct((NPAD, ET), _F32),
                   jax.ShapeDtypeStruct((NPAD, ET), _F32)],
    )(q0, q1, r, b, wa, wb, bfc)


def kernel(x, edges, W1, b1, W2, b2, Wfc, bfc):
    src = edges[0]
    dst = edges[1]
    xp = jnp.zeros((NPAD, D), _F32).at[:N].set(x)

    degp = _deg_sc(dst)
    s1 = _mm_tc(xp, W1)

    d0 = degp[0].reshape(NPAD, 1)
    d1 = degp[1].reshape(NPAD, 1)
    r, s1s = _rscale_tc(d0, d1, s1)

    qp = _segsum_sc(s1s, src, dst)
    s2s = _layer_mid_tc(qp[0], qp[1], r, b1.reshape(1, D), W2)

    qp2 = _segsum_sc(s2s, src, dst)
    p1, p2 = _layer_out_tc(qp2[0], qp2[1], r, b2.reshape(1, D),
                           Wfc[:D], Wfc[D:], bfc.reshape(1, ET))

    return _edge_mix_sc(p1, p2, src, dst)
